# baseline (speedup 1.0000x reference)
import functools

import jax
import jax.numpy as jnp
from jax import lax
from jax.experimental import pallas as pl
from jax.experimental.pallas import tpu as pltpu
from jax.experimental.pallas import tpu_sc as plsc

NC = 2
NS = 16
NW = NC * NS
BLK = 128


def _leaky(x):
    return jnp.where(x >= 0, x, 0.2 * x)


def _emb_body(cat_ref, w_ref, b_ref, out_ref):
    x = jnp.dot(cat_ref[...], w_ref[...], preferred_element_type=jnp.float32)
    out_ref[...] = _leaky(x + b_ref[0:1, :])


def _emb_call(cat, w, b2d):
    n, din = cat.shape
    dout = w.shape[1]
    bn = 2000
    return pl.pallas_call(
        _emb_body,
        grid=(n // bn,),
        in_specs=[
            pl.BlockSpec((bn, din), lambda i: (i, 0)),
            pl.BlockSpec((din, dout), lambda i: (0, 0)),
            pl.BlockSpec((8, dout), lambda i: (0, 0)),
        ],
        out_specs=pl.BlockSpec((bn, dout), lambda i: (i, 0)),
        out_shape=jax.ShapeDtypeStruct((n, dout), jnp.float32),
    )(cat, w, b2d)


def _make_edge_body(with_p, nch, f):
    def body(*refs):
        sg, og, pr, ws, wp, wo, b1a, w1b, b1b = refs[:9]
        t1 = _leaky(
            jnp.dot(sg[...], ws[...], preferred_element_type=jnp.float32)
            + jnp.dot(pr[...], wp[...], preferred_element_type=jnp.float32)
            + jnp.dot(og[...], wo[...], preferred_element_type=jnp.float32)
            + b1a[0:1, :])
        t2 = (jnp.dot(t1, w1b[...], preferred_element_type=jnp.float32)
              + b1b[0:1, :])
        h = nch * f
        if with_p:
            out_p = refs[9]
            d = out_p.shape[-1]
            out_p[...] = t2[:, h:h + d]
            s_refs = refs[10:10 + nch]
            o_refs = refs[10 + nch:]
            obase = h + d
        else:
            s_refs = refs[9:9 + nch]
            o_refs = refs[9 + nch:]
            obase = h
        for i in range(nch):
            s_refs[i][...] = t2[:, i * f:(i + 1) * f]
            o_refs[i][...] = t2[:, obase + i * f:obase + (i + 1) * f]
    return body


def _edge_call(sg, og, pr, ws, wp, wo, b1a2d, w1b, b1b2d, with_p, f):
    e, din = sg.shape
    dp = pr.shape[1]
    h = ws.shape[1]
    be = 1000
    kout = w1b.shape[1]
    do = kout - 2 * h
    nch = h // f
    row = lambda i: (i, 0)
    const = lambda i: (0, 0)
    in_specs = [
        pl.BlockSpec((be, din), row),
        pl.BlockSpec((be, din), row),
        pl.BlockSpec((be, dp), row),
        pl.BlockSpec((din, h), const),
        pl.BlockSpec((dp, h), const),
        pl.BlockSpec((din, h), const),
        pl.BlockSpec((8, h), const),
        pl.BlockSpec((h, kout), const),
        pl.BlockSpec((8, kout), const),
    ]
    chunk_shape = [jax.ShapeDtypeStruct((e, f), jnp.float32)] * nch
    chunk_spec = [pl.BlockSpec((be, f), row)] * nch
    if with_p:
        out_shape = [jax.ShapeDtypeStruct((e, do), jnp.float32)]
        out_specs = [pl.BlockSpec((be, do), row)]
    else:
        out_shape = []
        out_specs = []
    out_shape += chunk_shape + chunk_shape
    out_specs += chunk_spec + chunk_spec
    return pl.pallas_call(
        _make_edge_body(with_p, nch, f),
        grid=(e // be,),
        in_specs=in_specs,
        out_specs=out_specs,
        out_shape=out_shape,
    )(sg, og, pr, ws, wp, wo, b1a2d, w1b, b1b2d)


def _node_body(pp, cc, w2a, b2a, w2b, b2b, out):
    cnt = jnp.maximum(cc[0, :, 0:1] + cc[1, :, 0:1], 1.0)
    pooled = (pp[0] + pp[1]) / cnt
    h1 = _leaky(jnp.dot(pooled, w2a[...], preferred_element_type=jnp.float32)
                + b2a[0:1, :])
    out[...] = jnp.dot(h1, w2b[...], preferred_element_type=jnp.float32) + b2b[0:1, :]


def _node_body_final(pp, cc, w2a, b2a, w2b, b2b, wbb, bbb, out):
    cnt = jnp.maximum(cc[0, :, 0:1] + cc[1, :, 0:1], 1.0)
    pooled = (pp[0] + pp[1]) / cnt
    h1 = _leaky(jnp.dot(pooled, w2a[...], preferred_element_type=jnp.float32)
                + b2a[0:1, :])
    h2 = jnp.dot(h1, w2b[...], preferred_element_type=jnp.float32) + b2b[0:1, :]
    out[...] = _leaky(jnp.dot(h2, wbb[...], preferred_element_type=jnp.float32)
                      + bbb[0:1, :])


def _node_call(n, partials, counts, w2a, b2a2d, w2b, b2b2d, wbb, bbb2d):
    n2 = partials.shape[1]
    h = w2a.shape[0]
    do = w2b.shape[1]
    cf = counts.shape[2]
    bn = 2000
    row3 = lambda i: (0, i, 0)
    const = lambda i: (0, 0)
    in_specs = [
        pl.BlockSpec((2, bn, h), row3),
        pl.BlockSpec((2, bn, cf), row3),
        pl.BlockSpec((h, h), const),
        pl.BlockSpec((8, h), const),
        pl.BlockSpec((h, do), const),
        pl.BlockSpec((8, do), const),
    ]
    args = [partials, counts, w2a, b2a2d, w2b, b2b2d]
    if wbb is None:
        body = _node_body
        dout = do
    else:
        in_specs += [pl.BlockSpec((do, 128), const), pl.BlockSpec((8, 128), const)]
        args += [wbb, bbb2d]
        body = _node_body_final
        dout = 128
    return pl.pallas_call(
        body,
        grid=(n // bn,),
        in_specs=in_specs,
        out_specs=pl.BlockSpec((bn, dout), lambda i: (i, 0)),
        out_shape=jax.ShapeDtypeStruct((n, dout), jnp.float32),
    )(*args)


def _make_gather(n, d, e):
    nb = e // BLK
    bpt = nb // NW
    rem = nb - bpt * NW
    mesh = plsc.VectorSubcoreMesh(core_axis_name="c", subcore_axis_name="s")

    @functools.partial(
        pl.kernel,
        mesh=mesh,
        compiler_params=pltpu.CompilerParams(use_tc_tiling_on_sc=False),
        out_type=[jax.ShapeDtypeStruct((e, d), jnp.float32),
                  jax.ShapeDtypeStruct((e, d), jnp.float32)],
        scratch_types=[
            pltpu.VMEM((1, BLK), jnp.int32),
            pltpu.VMEM((1, BLK), jnp.int32),
            pltpu.VMEM((BLK, d), jnp.float32),
            pltpu.VMEM((BLK, d), jnp.float32),
            pltpu.SemaphoreType.DMA,
            pltpu.SemaphoreType.DMA,
        ],
    )
    def gat(table, si2, oi2, sg, og, idx_a, idx_b, rows_a, rows_b, asem, bsem):
        cid = lax.axis_index("c")
        sid = lax.axis_index("s")
        wid = sid * NC + cid
        b0 = wid * bpt
        my_nb = bpt + jnp.where(wid < rem, 1, 0)

        def blk_off(g):
            return jnp.where(g < bpt, b0 + g, NW * bpt + wid)

        pltpu.sync_copy(si2.at[pl.ds(blk_off(0), 1)], idx_a)
        pltpu.async_copy(table.at[idx_a.at[0]], rows_a, asem)

        def body(g, carry):
            bo = blk_off(g)
            ro = bo * BLK
            pltpu.make_async_copy(table.at[idx_a.at[0]], rows_a, asem).wait()
            pltpu.sync_copy(oi2.at[pl.ds(bo, 1)], idx_b)
            pltpu.async_copy(table.at[idx_b.at[0]], rows_b, bsem)
            pltpu.sync_copy(rows_a, sg.at[pl.ds(ro, BLK)])

            @pl.when(g + 1 < my_nb)
            def _():
                pltpu.sync_copy(si2.at[pl.ds(blk_off(g + 1), 1)], idx_a)
                pltpu.async_copy(table.at[idx_a.at[0]], rows_a, asem)

            pltpu.make_async_copy(table.at[idx_b.at[0]], rows_b, bsem).wait()
            pltpu.sync_copy(rows_b, og.at[pl.ds(ro, BLK)])
            return carry

        lax.fori_loop(0, my_nb, body, 0)

    return gat


def _make_scatter(e, n2, h, f):
    nb = e // BLK
    bpt = nb // NW
    rem = nb - bpt * NW
    npair = bpt // 2
    odd = bpt - 2 * npair
    nch = h // f
    stripe = n2 // NS
    mesh = plsc.VectorSubcoreMesh(core_axis_name="c", subcore_axis_name="s")

    @functools.partial(
        pl.kernel,
        mesh=mesh,
        compiler_params=pltpu.CompilerParams(use_tc_tiling_on_sc=False),
        out_type=jax.ShapeDtypeStruct((NC, n2, h), jnp.float32),
        scratch_types=[
            pltpu.VMEM_SHARED((n2, f), jnp.float32),
            pltpu.VMEM((2, BLK), jnp.int32),
            pltpu.VMEM((2, BLK), jnp.int32),
            pltpu.VMEM((2 * BLK, f), jnp.float32),
            pltpu.VMEM((2 * BLK, f), jnp.float32),
            pltpu.VMEM((BLK, f), jnp.float32),
            pltpu.SemaphoreType.DMA,
            pltpu.SemaphoreType.DMA,
        ],
    )
    def scat(*refs):
        vs_list = refs[:nch]
        vo_list = refs[nch:2 * nch]
        si2, oi2, out = refs[2 * nch:2 * nch + 3]
        acc, idx_a, idx_b, buf_a, buf_b, zero_v, asem, bsem = refs[2 * nch + 3:]
        cid = lax.axis_index("c")
        sid = lax.axis_index("s")
        wid = sid * NC + cid
        b0 = wid * bpt
        r0 = b0 * BLK

        def zfill(r, carry):
            for c16 in range(f // 16):
                zero_v[r, pl.ds(c16 * 16, 16)] = jnp.zeros((16,), jnp.float32)
            return carry

        lax.fori_loop(0, BLK, zfill, 0)

        for fc in range(nch):
            vs = vs_list[fc]
            vo = vo_list[fc]

            def zstripe(r, carry):
                pltpu.sync_copy(zero_v, acc.at[pl.ds(sid * stripe + r * BLK, BLK)])
                return carry

            lax.fori_loop(0, stripe // BLK, zstripe, 0)
            plsc.subcore_barrier()

            pltpu.async_copy(si2.at[pl.ds(b0, 2)], idx_a, asem)
            pltpu.async_copy(vs.at[pl.ds(r0, 2 * BLK)], buf_a, asem)

            def grp(g, carry):
                bo = b0 + 2 * g
                ro = bo * BLK
                pltpu.make_async_copy(si2.at[pl.ds(bo, 2)], idx_a, asem).wait()
                pltpu.make_async_copy(
                    vs.at[pl.ds(ro, 2 * BLK)], buf_a, asem).wait()
                pltpu.async_copy(oi2.at[pl.ds(bo, 2)], idx_b, bsem)
                pltpu.async_copy(vo.at[pl.ds(ro, 2 * BLK)], buf_b, bsem)
                pltpu.sync_copy(buf_a.at[pl.ds(0, BLK)],
                                acc.at[idx_a.at[0]], add=True)
                pltpu.sync_copy(buf_a.at[pl.ds(BLK, BLK)],
                                acc.at[idx_a.at[1]], add=True)
                pltpu.make_async_copy(oi2.at[pl.ds(bo, 2)], idx_b, bsem).wait()
                pltpu.make_async_copy(
                    vo.at[pl.ds(ro, 2 * BLK)], buf_b, bsem).wait()

                @pl.when(g + 1 < npair)
                def _():
                    bn = b0 + 2 * (g + 1)
                    pltpu.async_copy(si2.at[pl.ds(bn, 2)], idx_a, asem)
                    pltpu.async_copy(vs.at[pl.ds(bn * BLK, 2 * BLK)], buf_a,
                                     asem)

                pltpu.sync_copy(buf_b.at[pl.ds(0, BLK)],
                                acc.at[idx_b.at[0]], add=True)
                pltpu.sync_copy(buf_b.at[pl.ds(BLK, BLK)],
                                acc.at[idx_b.at[1]], add=True)
                return carry

            lax.fori_loop(0, npair, grp, 0)

            if odd:
                bo = b0 + 2 * npair
                pltpu.sync_copy(si2.at[pl.ds(bo, 1)], idx_a.at[pl.ds(0, 1)])
                pltpu.sync_copy(vs.at[pl.ds(bo * BLK, BLK)],
                                buf_a.at[pl.ds(0, BLK)])
                pltpu.sync_copy(buf_a.at[pl.ds(0, BLK)],
                                acc.at[idx_a.at[0]], add=True)
                pltpu.sync_copy(oi2.at[pl.ds(bo, 1)], idx_b.at[pl.ds(0, 1)])
                pltpu.sync_copy(vo.at[pl.ds(bo * BLK, BLK)],
                                buf_b.at[pl.ds(0, BLK)])
                pltpu.sync_copy(buf_b.at[pl.ds(0, BLK)],
                                acc.at[idx_b.at[0]], add=True)

            @pl.when(wid < rem)
            def _():
                bo = NW * bpt + wid
                pltpu.sync_copy(si2.at[pl.ds(bo, 1)], idx_a.at[pl.ds(0, 1)])
                pltpu.sync_copy(vs.at[pl.ds(bo * BLK, BLK)],
                                buf_a.at[pl.ds(0, BLK)])
                pltpu.sync_copy(buf_a.at[pl.ds(0, BLK)],
                                acc.at[idx_a.at[0]], add=True)
                pltpu.sync_copy(oi2.at[pl.ds(bo, 1)], idx_b.at[pl.ds(0, 1)])
                pltpu.sync_copy(vo.at[pl.ds(bo * BLK, BLK)],
                                buf_b.at[pl.ds(0, BLK)])
                pltpu.sync_copy(buf_b.at[pl.ds(0, BLK)],
                                acc.at[idx_b.at[0]], add=True)

            plsc.subcore_barrier()
            pltpu.sync_copy(
                acc.at[pl.ds(sid * stripe, stripe)],
                out.at[cid, pl.ds(sid * stripe, stripe), pl.ds(fc * f, f)])
            plsc.subcore_barrier()

    return scat


def _make_counts(e, n2):
    f = 16
    nb = e // BLK
    stripe = n2 // NS
    mesh = plsc.VectorSubcoreMesh(core_axis_name="c", subcore_axis_name="s")

    @functools.partial(
        pl.kernel,
        mesh=mesh,
        compiler_params=pltpu.CompilerParams(use_tc_tiling_on_sc=False),
        out_type=jax.ShapeDtypeStruct((NC, n2, f), jnp.float32),
        scratch_types=[
            pltpu.VMEM_SHARED((n2, f), jnp.float32),
            pltpu.VMEM((BLK,), jnp.int32),
            pltpu.VMEM((BLK, f), jnp.float32),
            pltpu.VMEM((BLK, f), jnp.float32),
        ],
    )
    def cnt(si, oi, out, acc, idx_v, ones_v, zero_v):
        cid = lax.axis_index("c")
        sid = lax.axis_index("s")
        wid = sid * NC + cid
        n_t = (nb - wid + NW - 1) // NW

        def fill(r, carry):
            ones_v[r, pl.ds(0, 16)] = jnp.full((16,), 1.0, jnp.float32)
            zero_v[r, pl.ds(0, 16)] = jnp.zeros((16,), jnp.float32)
            return carry

        lax.fori_loop(0, BLK, fill, 0)

        def zstripe(r, carry):
            pltpu.sync_copy(zero_v, acc.at[pl.ds(sid * stripe + r * BLK, BLK)])
            return carry

        lax.fori_loop(0, stripe // BLK, zstripe, 0)
        plsc.subcore_barrier()

        def eb(t, carry):
            off = (wid + NW * t) * BLK
            pltpu.sync_copy(si.at[pl.ds(off, BLK)], idx_v)
            pltpu.sync_copy(ones_v, acc.at[idx_v], add=True)
            pltpu.sync_copy(oi.at[pl.ds(off, BLK)], idx_v)
            pltpu.sync_copy(ones_v, acc.at[idx_v], add=True)
            return carry

        lax.fori_loop(0, n_t, eb, 0)
        plsc.subcore_barrier()
        pltpu.sync_copy(acc.at[pl.ds(sid * stripe, stripe)],
                        out.at[cid, pl.ds(sid * stripe, stripe)])
        plsc.subcore_barrier()

    return cnt


def _tile8(b):
    return jnp.tile(b[None, :], (8, 1))


def kernel(obj_vecs, pred_vecs, pred_boxes, s_idx, o_idx, params):
    n = obj_vecs.shape[0]
    e = pred_vecs.shape[0]
    n2 = ((n + NS * BLK - 1) // (NS * BLK)) * NS * BLK

    cat = jnp.concatenate(
        [obj_vecs, pred_boxes,
         jnp.zeros((n, 128 - obj_vecs.shape[1] - pred_boxes.shape[1]),
                   jnp.float32)], axis=1)
    wemb = jnp.concatenate(
        [params["W_emb"],
         jnp.zeros((128 - params["W_emb"].shape[0], params["W_emb"].shape[1]),
                   jnp.float32)], axis=0)
    x = _emb_call(cat, wemb, _tile8(params["b_emb"]))

    counts = _make_counts(e, n2)(s_idx, o_idx)
    si2 = s_idx.reshape(e // BLK, BLK)
    oi2 = o_idx.reshape(e // BLK, BLK)

    scatters = {}
    p = pred_vecs
    n_layers = 4
    for l in range(n_layers):
        w1a = params[f"W1a_{l}"]
        din = w1a.shape[0] // 3
        ws, wp, wo = w1a[:din], w1a[din:2 * din], w1a[2 * din:]
        w1b = params[f"W1b_{l}"]
        b1b = params[f"b1b_{l}"]
        h = w1a.shape[1]
        do = w1b.shape[1] - 2 * h
        last = l == n_layers - 1
        if last:
            w1b = jnp.concatenate([w1b[:, :h], w1b[:, h + do:]], axis=1)
            b1b = jnp.concatenate([b1b[:h], b1b[h + do:]], axis=0)
        if h not in scatters:
            scatters[h] = _make_scatter(e, n2, h, 32)
        scatter = scatters[h]

        gat = _make_gather(n, x.shape[1], e)
        sg, og = gat(x, si2, oi2)

        nch = h // 32
        outs = _edge_call(sg, og, p, ws, wp, wo, _tile8(params[f"b1a_{l}"]),
                          w1b, _tile8(b1b), with_p=not last, f=32)
        if last:
            s_list = outs[:nch]
            o_list = outs[nch:]
        else:
            p = outs[0]
            s_list = outs[1:1 + nch]
            o_list = outs[1 + nch:]

        partials = scatter(*s_list, *o_list, si2, oi2)

        if last:
            wbb = jnp.concatenate(
                [params["W_bb"],
                 jnp.zeros((do, 128 - params["W_bb"].shape[1]), jnp.float32)],
                axis=1)
            bbb = jnp.concatenate(
                [params["b_bb"],
                 jnp.zeros((128 - params["b_bb"].shape[0],), jnp.float32)],
                axis=0)
            x = _node_call(n, partials, counts, params[f"W2a_{l}"],
                           _tile8(params[f"b2a_{l}"]), params[f"W2b_{l}"],
                           _tile8(params[f"b2b_{l}"]), wbb, _tile8(bbb))
        else:
            x = _node_call(n, partials, counts, params[f"W2a_{l}"],
                           _tile8(params[f"b2a_{l}"]), params[f"W2b_{l}"],
                           _tile8(params[f"b2b_{l}"]), None, None)

    return x[:, :params["W_bb"].shape[1]]


# speedup vs baseline: 1.9987x; 1.9987x over previous
import functools

import jax
import jax.numpy as jnp
from jax import lax
from jax.experimental import pallas as pl
from jax.experimental.pallas import tpu as pltpu
from jax.experimental.pallas import tpu_sc as plsc

NC = 2
NS = 16
NW = NC * NS
BLK = 128


def _leaky(x):
    return jnp.where(x >= 0, x, 0.2 * x)


def _emb_body(cat_ref, w_ref, b_ref, out_ref):
    x = jnp.dot(cat_ref[...], w_ref[...], preferred_element_type=jnp.float32)
    out_ref[...] = _leaky(x + b_ref[0:1, :])


def _emb_call(cat, w, b2d):
    n, din = cat.shape
    dout = w.shape[1]
    bn = 2000
    return pl.pallas_call(
        _emb_body,
        grid=(n // bn,),
        in_specs=[
            pl.BlockSpec((bn, din), lambda i: (i, 0)),
            pl.BlockSpec((din, dout), lambda i: (0, 0)),
            pl.BlockSpec((8, dout), lambda i: (0, 0)),
        ],
        out_specs=pl.BlockSpec((bn, dout), lambda i: (i, 0)),
        out_shape=jax.ShapeDtypeStruct((n, dout), jnp.float32),
    )(cat, w, b2d)


def _edge_body_p(sg, og, pr, ws, wp, wo, b1a, w1b, b1b, out_s, out_p, out_o):
    t1 = _leaky(
        jnp.dot(sg[...], ws[...], preferred_element_type=jnp.float32)
        + jnp.dot(pr[...], wp[...], preferred_element_type=jnp.float32)
        + jnp.dot(og[...], wo[...], preferred_element_type=jnp.float32)
        + b1a[0:1, :])
    t2 = jnp.dot(t1, w1b[...], preferred_element_type=jnp.float32) + b1b[0:1, :]
    h = out_s.shape[-1]
    d = out_p.shape[-1]
    out_s[...] = t2[:, :h]
    out_p[...] = t2[:, h:h + d]
    out_o[...] = t2[:, h + d:]


def _edge_body_np(sg, og, pr, ws, wp, wo, b1a, w1b, b1b, out_s, out_o):
    t1 = _leaky(
        jnp.dot(sg[...], ws[...], preferred_element_type=jnp.float32)
        + jnp.dot(pr[...], wp[...], preferred_element_type=jnp.float32)
        + jnp.dot(og[...], wo[...], preferred_element_type=jnp.float32)
        + b1a[0:1, :])
    t2 = jnp.dot(t1, w1b[...], preferred_element_type=jnp.float32) + b1b[0:1, :]
    h = out_s.shape[-1]
    out_s[...] = t2[:, :h]
    out_o[...] = t2[:, h:]


def _edge_call(sg, og, pr, ws, wp, wo, b1a2d, w1b, b1b2d, with_p):
    e, din = sg.shape
    dp = pr.shape[1]
    h = ws.shape[1]
    be = 2000
    kout = w1b.shape[1]
    do = kout - 2 * h
    row = lambda i: (i, 0)
    const = lambda i: (0, 0)
    in_specs = [
        pl.BlockSpec((be, din), row),
        pl.BlockSpec((be, din), row),
        pl.BlockSpec((be, dp), row),
        pl.BlockSpec((din, h), const),
        pl.BlockSpec((dp, h), const),
        pl.BlockSpec((din, h), const),
        pl.BlockSpec((8, h), const),
        pl.BlockSpec((h, kout), const),
        pl.BlockSpec((8, kout), const),
    ]
    if with_p:
        out_shape = [
            jax.ShapeDtypeStruct((e, h), jnp.float32),
            jax.ShapeDtypeStruct((e, do), jnp.float32),
            jax.ShapeDtypeStruct((e, h), jnp.float32),
        ]
        out_specs = [
            pl.BlockSpec((be, h), row),
            pl.BlockSpec((be, do), row),
            pl.BlockSpec((be, h), row),
        ]
        body = _edge_body_p
    else:
        out_shape = [
            jax.ShapeDtypeStruct((e, h), jnp.float32),
            jax.ShapeDtypeStruct((e, h), jnp.float32),
        ]
        out_specs = [
            pl.BlockSpec((be, h), row),
            pl.BlockSpec((be, h), row),
        ]
        body = _edge_body_np
    return pl.pallas_call(
        body,
        grid=(e // be,),
        in_specs=in_specs,
        out_specs=out_specs,
        out_shape=out_shape,
    )(sg, og, pr, ws, wp, wo, b1a2d, w1b, b1b2d)


def _node_body(pp, cc, w2a, b2a, w2b, b2b, out):
    cnt = jnp.maximum(cc[0, :, 0:1] + cc[1, :, 0:1], 1.0)
    pooled = (pp[0] + pp[1]) / cnt
    h1 = _leaky(jnp.dot(pooled, w2a[...], preferred_element_type=jnp.float32)
                + b2a[0:1, :])
    out[...] = jnp.dot(h1, w2b[...], preferred_element_type=jnp.float32) + b2b[0:1, :]


def _node_body_final(pp, cc, w2a, b2a, w2b, b2b, wbb, bbb, out):
    cnt = jnp.maximum(cc[0, :, 0:1] + cc[1, :, 0:1], 1.0)
    pooled = (pp[0] + pp[1]) / cnt
    h1 = _leaky(jnp.dot(pooled, w2a[...], preferred_element_type=jnp.float32)
                + b2a[0:1, :])
    h2 = jnp.dot(h1, w2b[...], preferred_element_type=jnp.float32) + b2b[0:1, :]
    out[...] = _leaky(jnp.dot(h2, wbb[...], preferred_element_type=jnp.float32)
                      + bbb[0:1, :])


def _node_call(n, partials, counts, w2a, b2a2d, w2b, b2b2d, wbb, bbb2d):
    n2 = partials.shape[1]
    h = w2a.shape[0]
    do = w2b.shape[1]
    cf = counts.shape[2]
    bn = 2000
    row3 = lambda i: (0, i, 0)
    const = lambda i: (0, 0)
    in_specs = [
        pl.BlockSpec((2, bn, h), row3),
        pl.BlockSpec((2, bn, cf), row3),
        pl.BlockSpec((h, h), const),
        pl.BlockSpec((8, h), const),
        pl.BlockSpec((h, do), const),
        pl.BlockSpec((8, do), const),
    ]
    args = [partials, counts, w2a, b2a2d, w2b, b2b2d]
    if wbb is None:
        body = _node_body
        dout = do
    else:
        in_specs += [pl.BlockSpec((do, 128), const), pl.BlockSpec((8, 128), const)]
        args += [wbb, bbb2d]
        body = _node_body_final
        dout = 128
    return pl.pallas_call(
        body,
        grid=(n // bn,),
        in_specs=in_specs,
        out_specs=pl.BlockSpec((bn, dout), lambda i: (i, 0)),
        out_shape=jax.ShapeDtypeStruct((n, dout), jnp.float32),
    )(*args)


def _make_gather(n, d, e):
    nb = e // BLK
    bpt = nb // NW
    rem = nb - bpt * NW
    mesh = plsc.VectorSubcoreMesh(core_axis_name="c", subcore_axis_name="s")

    @functools.partial(
        pl.kernel,
        mesh=mesh,
        compiler_params=pltpu.CompilerParams(use_tc_tiling_on_sc=False),
        out_type=[jax.ShapeDtypeStruct((e, d), jnp.float32),
                  jax.ShapeDtypeStruct((e, d), jnp.float32)],
        scratch_types=[
            pltpu.VMEM((1, BLK), jnp.int32),
            pltpu.VMEM((1, BLK), jnp.int32),
            pltpu.VMEM((BLK, d), jnp.float32),
            pltpu.VMEM((BLK, d), jnp.float32),
            pltpu.SemaphoreType.DMA,
            pltpu.SemaphoreType.DMA,
        ],
    )
    def gat(table, si2, oi2, sg, og, idx_a, idx_b, rows_a, rows_b, asem, bsem):
        cid = lax.axis_index("c")
        sid = lax.axis_index("s")
        wid = sid * NC + cid
        b0 = wid * bpt
        my_nb = bpt + jnp.where(wid < rem, 1, 0)

        def blk_off(g):
            return jnp.where(g < bpt, b0 + g, NW * bpt + wid)

        pltpu.sync_copy(si2.at[pl.ds(blk_off(0), 1)], idx_a)
        pltpu.async_copy(table.at[idx_a.at[0]], rows_a, asem)

        def body(g, carry):
            bo = blk_off(g)
            ro = bo * BLK
            pltpu.make_async_copy(table.at[idx_a.at[0]], rows_a, asem).wait()
            pltpu.sync_copy(oi2.at[pl.ds(bo, 1)], idx_b)
            pltpu.async_copy(table.at[idx_b.at[0]], rows_b, bsem)
            pltpu.sync_copy(rows_a, sg.at[pl.ds(ro, BLK)])

            @pl.when(g + 1 < my_nb)
            def _():
                pltpu.sync_copy(si2.at[pl.ds(blk_off(g + 1), 1)], idx_a)
                pltpu.async_copy(table.at[idx_a.at[0]], rows_a, asem)

            pltpu.make_async_copy(table.at[idx_b.at[0]], rows_b, bsem).wait()
            pltpu.sync_copy(rows_b, og.at[pl.ds(ro, BLK)])
            return carry

        lax.fori_loop(0, my_nb, body, 0)

    return gat


def _make_scatter(e, n2, h, f):
    nb = e // BLK
    bpt = nb // NW
    rem = nb - bpt * NW
    npair = bpt // 2
    odd = bpt - 2 * npair
    nch = h // f
    stripe = n2 // NS
    mesh = plsc.VectorSubcoreMesh(core_axis_name="c", subcore_axis_name="s")

    @functools.partial(
        pl.kernel,
        mesh=mesh,
        compiler_params=pltpu.CompilerParams(use_tc_tiling_on_sc=False),
        out_type=jax.ShapeDtypeStruct((NC, n2, h), jnp.float32),
        scratch_types=[
            pltpu.VMEM_SHARED((n2, f), jnp.float32),
            pltpu.VMEM((2, BLK), jnp.int32),
            pltpu.VMEM((2, BLK), jnp.int32),
            pltpu.VMEM((2 * BLK, f), jnp.float32),
            pltpu.VMEM((2 * BLK, f), jnp.float32),
            pltpu.VMEM((BLK, f), jnp.float32),
            pltpu.SemaphoreType.DMA,
            pltpu.SemaphoreType.DMA,
        ],
    )
    def scat(vs, vo, si2, oi2, out, acc, idx_a, idx_b, buf_a, buf_b, zero_v,
             asem, bsem):
        cid = lax.axis_index("c")
        sid = lax.axis_index("s")
        wid = sid * NC + cid
        b0 = wid * bpt
        r0 = b0 * BLK

        def zfill(r, carry):
            for c16 in range(f // 16):
                zero_v[r, pl.ds(c16 * 16, 16)] = jnp.zeros((16,), jnp.float32)
            return carry

        lax.fori_loop(0, BLK, zfill, 0)

        def chunk(fc, c0):
            col = fc * f

            def zstripe(r, carry):
                pltpu.sync_copy(zero_v, acc.at[pl.ds(sid * stripe + r * BLK, BLK)])
                return carry

            lax.fori_loop(0, stripe // BLK, zstripe, 0)
            plsc.subcore_barrier()

            pltpu.async_copy(si2.at[pl.ds(b0, 2)], idx_a, asem)
            pltpu.async_copy(vs.at[pl.ds(r0, 2 * BLK), pl.ds(col, f)], buf_a,
                             asem)

            def grp(g, carry):
                bo = b0 + 2 * g
                ro = bo * BLK
                pltpu.make_async_copy(si2.at[pl.ds(bo, 2)], idx_a, asem).wait()
                pltpu.make_async_copy(
                    vs.at[pl.ds(ro, 2 * BLK), pl.ds(col, f)], buf_a,
                    asem).wait()
                pltpu.async_copy(oi2.at[pl.ds(bo, 2)], idx_b, bsem)
                pltpu.async_copy(vo.at[pl.ds(ro, 2 * BLK), pl.ds(col, f)],
                                 buf_b, bsem)
                pltpu.sync_copy(buf_a.at[pl.ds(0, BLK)],
                                acc.at[idx_a.at[0]], add=True)
                pltpu.sync_copy(buf_a.at[pl.ds(BLK, BLK)],
                                acc.at[idx_a.at[1]], add=True)
                pltpu.make_async_copy(oi2.at[pl.ds(bo, 2)], idx_b, bsem).wait()
                pltpu.make_async_copy(
                    vo.at[pl.ds(ro, 2 * BLK), pl.ds(col, f)], buf_b,
                    bsem).wait()

                @pl.when(g + 1 < npair)
                def _():
                    bn = b0 + 2 * (g + 1)
                    pltpu.async_copy(si2.at[pl.ds(bn, 2)], idx_a, asem)
                    pltpu.async_copy(
                        vs.at[pl.ds(bn * BLK, 2 * BLK), pl.ds(col, f)], buf_a,
                        asem)

                pltpu.sync_copy(buf_b.at[pl.ds(0, BLK)],
                                acc.at[idx_b.at[0]], add=True)
                pltpu.sync_copy(buf_b.at[pl.ds(BLK, BLK)],
                                acc.at[idx_b.at[1]], add=True)
                return carry

            lax.fori_loop(0, npair, grp, 0)

            if odd:
                bo = b0 + 2 * npair
                pltpu.sync_copy(si2.at[pl.ds(bo, 1)], idx_a.at[pl.ds(0, 1)])
                pltpu.sync_copy(vs.at[pl.ds(bo * BLK, BLK), pl.ds(col, f)],
                                buf_a.at[pl.ds(0, BLK)])
                pltpu.sync_copy(buf_a.at[pl.ds(0, BLK)],
                                acc.at[idx_a.at[0]], add=True)
                pltpu.sync_copy(oi2.at[pl.ds(bo, 1)], idx_b.at[pl.ds(0, 1)])
                pltpu.sync_copy(vo.at[pl.ds(bo * BLK, BLK), pl.ds(col, f)],
                                buf_b.at[pl.ds(0, BLK)])
                pltpu.sync_copy(buf_b.at[pl.ds(0, BLK)],
                                acc.at[idx_b.at[0]], add=True)

            @pl.when(wid < rem)
            def _():
                bo = NW * bpt + wid
                pltpu.sync_copy(si2.at[pl.ds(bo, 1)], idx_a.at[pl.ds(0, 1)])
                pltpu.sync_copy(vs.at[pl.ds(bo * BLK, BLK), pl.ds(col, f)],
                                buf_a.at[pl.ds(0, BLK)])
                pltpu.sync_copy(buf_a.at[pl.ds(0, BLK)],
                                acc.at[idx_a.at[0]], add=True)
                pltpu.sync_copy(oi2.at[pl.ds(bo, 1)], idx_b.at[pl.ds(0, 1)])
                pltpu.sync_copy(vo.at[pl.ds(bo * BLK, BLK), pl.ds(col, f)],
                                buf_b.at[pl.ds(0, BLK)])
                pltpu.sync_copy(buf_b.at[pl.ds(0, BLK)],
                                acc.at[idx_b.at[0]], add=True)

            plsc.subcore_barrier()
            pltpu.sync_copy(
                acc.at[pl.ds(sid * stripe, stripe)],
                out.at[cid, pl.ds(sid * stripe, stripe), pl.ds(col, f)])
            plsc.subcore_barrier()
            return c0

        lax.fori_loop(0, nch, chunk, 0)

    return scat


def _make_counts(e, n2):
    f = 16
    nb = e // BLK
    stripe = n2 // NS
    mesh = plsc.VectorSubcoreMesh(core_axis_name="c", subcore_axis_name="s")

    @functools.partial(
        pl.kernel,
        mesh=mesh,
        compiler_params=pltpu.CompilerParams(use_tc_tiling_on_sc=False),
        out_type=jax.ShapeDtypeStruct((NC, n2, f), jnp.float32),
        scratch_types=[
            pltpu.VMEM_SHARED((n2, f), jnp.float32),
            pltpu.VMEM((BLK,), jnp.int32),
            pltpu.VMEM((BLK, f), jnp.float32),
            pltpu.VMEM((BLK, f), jnp.float32),
        ],
    )
    def cnt(si, oi, out, acc, idx_v, ones_v, zero_v):
        cid = lax.axis_index("c")
        sid = lax.axis_index("s")
        wid = sid * NC + cid
        n_t = (nb - wid + NW - 1) // NW

        def fill(r, carry):
            ones_v[r, pl.ds(0, 16)] = jnp.full((16,), 1.0, jnp.float32)
            zero_v[r, pl.ds(0, 16)] = jnp.zeros((16,), jnp.float32)
            return carry

        lax.fori_loop(0, BLK, fill, 0)

        def zstripe(r, carry):
            pltpu.sync_copy(zero_v, acc.at[pl.ds(sid * stripe + r * BLK, BLK)])
            return carry

        lax.fori_loop(0, stripe // BLK, zstripe, 0)
        plsc.subcore_barrier()

        def eb(t, carry):
            off = (wid + NW * t) * BLK
            pltpu.sync_copy(si.at[pl.ds(off, BLK)], idx_v)
            pltpu.sync_copy(ones_v, acc.at[idx_v], add=True)
            pltpu.sync_copy(oi.at[pl.ds(off, BLK)], idx_v)
            pltpu.sync_copy(ones_v, acc.at[idx_v], add=True)
            return carry

        lax.fori_loop(0, n_t, eb, 0)
        plsc.subcore_barrier()
        pltpu.sync_copy(acc.at[pl.ds(sid * stripe, stripe)],
                        out.at[cid, pl.ds(sid * stripe, stripe)])
        plsc.subcore_barrier()

    return cnt


def _tile8(b):
    return jnp.tile(b[None, :], (8, 1))


def kernel(obj_vecs, pred_vecs, pred_boxes, s_idx, o_idx, params):
    n = obj_vecs.shape[0]
    e = pred_vecs.shape[0]
    n2 = ((n + NS * BLK - 1) // (NS * BLK)) * NS * BLK

    cat = jnp.concatenate(
        [obj_vecs, pred_boxes,
         jnp.zeros((n, 128 - obj_vecs.shape[1] - pred_boxes.shape[1]),
                   jnp.float32)], axis=1)
    wemb = jnp.concatenate(
        [params["W_emb"],
         jnp.zeros((128 - params["W_emb"].shape[0], params["W_emb"].shape[1]),
                   jnp.float32)], axis=0)
    x = _emb_call(cat, wemb, _tile8(params["b_emb"]))

    counts = _make_counts(e, n2)(s_idx, o_idx)
    si2 = s_idx.reshape(e // BLK, BLK)
    oi2 = o_idx.reshape(e // BLK, BLK)

    scatters = {}
    p = pred_vecs
    n_layers = 4
    for l in range(n_layers):
        w1a = params[f"W1a_{l}"]
        din = w1a.shape[0] // 3
        ws, wp, wo = w1a[:din], w1a[din:2 * din], w1a[2 * din:]
        w1b = params[f"W1b_{l}"]
        b1b = params[f"b1b_{l}"]
        h = w1a.shape[1]
        do = w1b.shape[1] - 2 * h
        last = l == n_layers - 1
        if last:
            w1b = jnp.concatenate([w1b[:, :h], w1b[:, h + do:]], axis=1)
            b1b = jnp.concatenate([b1b[:h], b1b[h + do:]], axis=0)
        if h not in scatters:
            scatters[h] = _make_scatter(e, n2, h, 32)
        scatter = scatters[h]

        gat = _make_gather(n, x.shape[1], e)
        sg, og = gat(x, si2, oi2)

        outs = _edge_call(sg, og, p, ws, wp, wo, _tile8(params[f"b1a_{l}"]),
                          w1b, _tile8(b1b), with_p=not last)
        if last:
            new_s, new_o = outs
        else:
            new_s, p, new_o = outs

        partials = scatter(new_s, new_o, si2, oi2)

        if last:
            wbb = jnp.concatenate(
                [params["W_bb"],
                 jnp.zeros((do, 128 - params["W_bb"].shape[1]), jnp.float32)],
                axis=1)
            bbb = jnp.concatenate(
                [params["b_bb"],
                 jnp.zeros((128 - params["b_bb"].shape[0],), jnp.float32)],
                axis=0)
            x = _node_call(n, partials, counts, params[f"W2a_{l}"],
                           _tile8(params[f"b2a_{l}"]), params[f"W2b_{l}"],
                           _tile8(params[f"b2b_{l}"]), wbb, _tile8(bbb))
        else:
            x = _node_call(n, partials, counts, params[f"W2a_{l}"],
                           _tile8(params[f"b2a_{l}"]), params[f"W2b_{l}"],
                           _tile8(params[f"b2b_{l}"]), None, None)

    return x[:, :params["W_bb"].shape[1]]
